# Initial kernel scaffold; baseline (speedup 1.0000x reference)
import functools

import jax
import jax.numpy as jnp
from jax import lax
from jax.experimental import pallas as pl
from jax.experimental.pallas import tpu as pltpu
from jax.experimental.pallas import tpu_sc as plsc

N = 10000
E = 320000
D = 128
NC = 2
NS = 16
C = 80
EPW = E // (NC * NS)
CHUNKS = EPW // C
RPT = N // NS


def _sc_aggregate(h, src, dst, zeros):
    mesh = plsc.VectorSubcoreMesh(core_axis_name="c", subcore_axis_name="s")

    @functools.partial(
        pl.kernel,
        mesh=mesh,
        out_type=jax.ShapeDtypeStruct((NC, N, D), jnp.float32),
        scratch_types=[
            pltpu.VMEM((C,), jnp.int32),
            pltpu.VMEM((C,), jnp.int32),
            pltpu.VMEM((C, D), jnp.float32),
            pltpu.VMEM_SHARED((N, D), jnp.float32),
            pltpu.SemaphoreType.DMA,
        ],
    )
    def agg_kernel(h_hbm, src_hbm, dst_hbm, zeros_hbm, out_hbm,
                   src_v, dst_v, rows_v, acc_sh, sem):
        c = lax.axis_index("c")
        s = lax.axis_index("s")
        row0 = s * RPT

        @pl.when(c == 0)
        def _():
            pltpu.sync_copy(h_hbm.at[pl.ds(row0, RPT)],
                            acc_sh.at[pl.ds(row0, RPT)])

        @pl.when(c != 0)
        def _():
            pltpu.sync_copy(zeros_hbm.at[pl.ds(row0, RPT)],
                            acc_sh.at[pl.ds(row0, RPT)])

        plsc.subcore_barrier()

        base = (c * NS + s) * EPW

        def body(g, carry):
            eb = pl.multiple_of(base + g * C, 8)
            pltpu.sync_copy(src_hbm.at[pl.ds(eb, C)], src_v)
            pltpu.sync_copy(dst_hbm.at[pl.ds(eb, C)], dst_v)
            pltpu.async_copy(h_hbm.at[src_v], rows_v, sem).wait()
            pltpu.sync_copy(rows_v, acc_sh.at[dst_v], add=True)
            return carry

        lax.fori_loop(0, CHUNKS, body, 0)

        plsc.subcore_barrier()
        pltpu.sync_copy(acc_sh.at[pl.ds(row0, RPT)],
                        out_hbm.at[c, pl.ds(row0, RPT)])

    return agg_kernel(h, src, dst, zeros)


def _tc_mlp(agg, W, b):
    d_out = W.shape[1]
    BR = 1000

    def mm_kernel(a_ref, w_ref, b_ref, o_ref):
        x = a_ref[0] + a_ref[1]
        o_ref[...] = jnp.dot(x, w_ref[...],
                             preferred_element_type=jnp.float32) + b_ref[...]

    return pl.pallas_call(
        mm_kernel,
        grid=(N // BR,),
        in_specs=[
            pl.BlockSpec((2, BR, D), lambda i: (0, i, 0)),
            pl.BlockSpec((D, d_out), lambda i: (0, 0)),
            pl.BlockSpec((1, d_out), lambda i: (0, 0)),
        ],
        out_specs=pl.BlockSpec((BR, d_out), lambda i: (i, 0)),
        out_shape=jax.ShapeDtypeStruct((N, d_out), jnp.float32),
    )(agg, W, b.reshape(1, d_out))


def kernel(features, edge_index, W_in, b_in, W_hid, b_hid, W_out, b_out):
    src = edge_index[0]
    dst = edge_index[1]
    zeros = jnp.zeros((N, D), jnp.float32)
    h = features
    for W, b in ((W_in, b_in), (W_hid, b_hid), (W_out, b_out)):
        agg = _sc_aggregate(h, src, dst, zeros)
        h = _tc_mlp(agg, W, b)
    return h


# speedup vs baseline: 4.5238x; 4.5238x over previous
import functools

import jax
import jax.numpy as jnp
from jax import lax
from jax.experimental import pallas as pl
from jax.experimental.pallas import tpu as pltpu
from jax.experimental.pallas import tpu_sc as plsc

N = 10000
E = 320000
D = 128
NC = 2
NS = 16
C = 80
EPW = E // (NC * NS)
CHUNKS = EPW // C
RPT = 624
TAIL0 = RPT * NS
TAIL = N - TAIL0


def _sc_aggregate(h, src, dst, zeros):
    mesh = plsc.VectorSubcoreMesh(core_axis_name="c", subcore_axis_name="s")

    @functools.partial(
        pl.kernel,
        mesh=mesh,
        out_type=jax.ShapeDtypeStruct((NC, N, D), jnp.float32),
        scratch_types=[
            pltpu.VMEM((C,), jnp.int32),
            pltpu.VMEM((C,), jnp.int32),
            pltpu.VMEM((C, D), jnp.float32),
            pltpu.VMEM_SHARED((N, D), jnp.float32),
            pltpu.SemaphoreType.DMA,
        ],
    )
    def agg_kernel(h_hbm, src_hbm, dst_hbm, zeros_hbm, out_hbm,
                   src_v, dst_v, rows_v, acc_sh, sem):
        c = lax.axis_index("c")
        s = lax.axis_index("s")
        row0 = s * RPT

        @pl.when(c == 0)
        def _():
            pltpu.sync_copy(h_hbm.at[pl.ds(row0, RPT)],
                            acc_sh.at[pl.ds(row0, RPT)])

            @pl.when(s == NS - 1)
            def _():
                pltpu.sync_copy(h_hbm.at[pl.ds(TAIL0, TAIL)],
                                acc_sh.at[pl.ds(TAIL0, TAIL)])

        @pl.when(c != 0)
        def _():
            pltpu.sync_copy(zeros_hbm.at[pl.ds(row0, RPT)],
                            acc_sh.at[pl.ds(row0, RPT)])

            @pl.when(s == NS - 1)
            def _():
                pltpu.sync_copy(zeros_hbm.at[pl.ds(TAIL0, TAIL)],
                                acc_sh.at[pl.ds(TAIL0, TAIL)])

        plsc.subcore_barrier()

        base = (c * NS + s) * EPW

        def body(g, carry):
            eb = pl.multiple_of(base + g * C, 8)
            pltpu.sync_copy(src_hbm.at[pl.ds(eb, C)], src_v)
            pltpu.sync_copy(dst_hbm.at[pl.ds(eb, C)], dst_v)
            pltpu.async_copy(h_hbm.at[src_v], rows_v, sem).wait()
            pltpu.sync_copy(rows_v, acc_sh.at[dst_v], add=True)
            return carry

        lax.fori_loop(0, CHUNKS, body, 0)

        plsc.subcore_barrier()
        pltpu.sync_copy(acc_sh.at[pl.ds(row0, RPT)],
                        out_hbm.at[c, pl.ds(row0, RPT)])

        @pl.when(s == NS - 1)
        def _():
            pltpu.sync_copy(acc_sh.at[pl.ds(TAIL0, TAIL)],
                            out_hbm.at[c, pl.ds(TAIL0, TAIL)])

    return agg_kernel(h, src, dst, zeros)


def _tc_mlp(agg, W, b):
    d_out = W.shape[1]
    BR = 1000

    def mm_kernel(a_ref, w_ref, b_ref, o_ref):
        x = a_ref[0] + a_ref[1]
        o_ref[...] = jnp.dot(x, w_ref[...],
                             preferred_element_type=jnp.float32) + b_ref[...]

    return pl.pallas_call(
        mm_kernel,
        grid=(N // BR,),
        in_specs=[
            pl.BlockSpec((2, BR, D), lambda i: (0, i, 0)),
            pl.BlockSpec((D, d_out), lambda i: (0, 0)),
            pl.BlockSpec((1, d_out), lambda i: (0, 0)),
        ],
        out_specs=pl.BlockSpec((BR, d_out), lambda i: (i, 0)),
        out_shape=jax.ShapeDtypeStruct((N, d_out), jnp.float32),
    )(agg, W, b.reshape(1, d_out))


def kernel(features, edge_index, W_in, b_in, W_hid, b_hid, W_out, b_out):
    src = edge_index[0]
    dst = edge_index[1]
    zeros = jnp.zeros((N, D), jnp.float32)
    h = features
    for W, b in ((W_in, b_in), (W_hid, b_hid), (W_out, b_out)):
        agg = _sc_aggregate(h, src, dst, zeros)
        h = _tc_mlp(agg, W, b)
    return h
